# Initial kernel scaffold; baseline (speedup 1.0000x reference)
import jax
import jax.numpy as jnp
from jax.experimental import pallas as pl


def kernel(img, weight):
    raise NotImplementedError("write your pallas kernel here")


# speedup vs baseline: 1.4606x; 1.4606x over previous
import functools

import jax
import jax.numpy as jnp
from jax import lax
from jax.experimental import pallas as pl
from jax.experimental.pallas import tpu as pltpu
from jax.experimental.pallas import tpu_sc as plsc

W = 224
K = 128
M = 8 * 192 * 224
NC = 2
NS = 16
NW = NC * NS
ROWS_PER_W = M // NW
R = 128
CHUNKS = ROWS_PER_W // R


def _topk_body(w_ref, ind_ref):
    w = w_ref[...]
    s_row = jnp.sum(w, axis=0, keepdims=True)
    ii = lax.broadcasted_iota(jnp.int32, (W, W), 0)
    jj = lax.broadcasted_iota(jnp.int32, (W, W), 1)
    eye = (ii == jj).astype(jnp.float32)
    s_col = lax.dot_general(eye, s_row,
                            dimension_numbers=(((1,), (1,)), ((), ())),
                            preferred_element_type=jnp.float32)
    lane = lax.broadcasted_iota(jnp.int32, (W, W), 1)
    sub = lax.broadcasted_iota(jnp.int32, (W, W), 0)
    lt = (s_row < s_col).astype(jnp.float32)
    tie = ((s_row == s_col) & (lane < sub)).astype(jnp.float32)
    rank = jnp.sum(lt + tie, axis=1, keepdims=True)
    j = rank - float(W - K)
    jlane = lax.broadcasted_iota(jnp.int32, (W, K), 1).astype(jnp.float32)
    onehot = (j == jlane).astype(jnp.float32)
    csub = lax.broadcasted_iota(jnp.int32, (W, K), 0).astype(jnp.float32)
    ind_f = jnp.sum(onehot * csub, axis=0, keepdims=True)
    ind_ref[...] = ind_f.astype(jnp.int32)


_topk = pl.pallas_call(
    _topk_body,
    out_shape=jax.ShapeDtypeStruct((1, K), jnp.int32),
)


def _sc_gather_body(img_hbm, ind_hbm, out_hbm, ind_v, idx_v, in_v, out_v):
    wid = lax.axis_index("s") * NC + lax.axis_index("c")
    base_row = wid * ROWS_PER_W

    pltpu.sync_copy(ind_hbm, ind_v)

    def prerow(r, carry):
        for jv in range(K // 16):
            iv = ind_v[pl.ds(16 * jv, 16)]
            idx_v[pl.ds(r * K + 16 * jv, 16)] = iv + r * W
        return carry

    lax.fori_loop(0, R, prerow, 0)

    def chunk_body(c, carry):
        row0 = base_row + c * R
        pltpu.sync_copy(img_hbm.at[pl.ds(row0 * W, R * W)], in_v)

        def t_body(t, carry2):
            idx = idx_v[pl.ds(t * 16, 16)]
            out_v[pl.ds(t * 16, 16)] = plsc.load_gather(in_v, [idx])
            return carry2

        lax.fori_loop(0, R * (K // 16), t_body, 0)
        pltpu.sync_copy(out_v, out_hbm.at[pl.ds(row0 * K, R * K)])
        return carry

    lax.fori_loop(0, CHUNKS, chunk_body, 0)


_sc_gather = functools.partial(
    pl.kernel,
    mesh=plsc.VectorSubcoreMesh(core_axis_name="c", subcore_axis_name="s"),
    out_type=jax.ShapeDtypeStruct((M * K,), jnp.float32),
    compiler_params=pltpu.CompilerParams(needs_layout_passes=False),
    scratch_types=[
        pltpu.VMEM((K,), jnp.int32),
        pltpu.VMEM((R * K,), jnp.int32),
        pltpu.VMEM((R * W,), jnp.float32),
        pltpu.VMEM((R * K,), jnp.float32),
    ],
)(_sc_gather_body)


def kernel(img, weight):
    w2 = weight[:, :, 0, 0]
    ind = _topk(w2).reshape(K)
    img_flat = img.reshape(M * W)
    out_flat = _sc_gather(img_flat, ind)
    return out_flat.reshape(8, 192, 224, K)


# speedup vs baseline: 2.9118x; 1.9936x over previous
import functools

import jax
import jax.numpy as jnp
from jax import lax
from jax.experimental import pallas as pl
from jax.experimental.pallas import tpu as pltpu
from jax.experimental.pallas import tpu_sc as plsc

W = 224
K = 128
M = 8 * 192 * 224
NC = 2
NS = 16
NW = NC * NS
ROWS_PER_W = M // NW
R = 168
CHUNKS = ROWS_PER_W // R


def _topk_body(w_ref, ind_ref):
    w = w_ref[...]
    s_row = jnp.sum(w, axis=0, keepdims=True)
    ii = lax.broadcasted_iota(jnp.int32, (W, W), 0)
    jj = lax.broadcasted_iota(jnp.int32, (W, W), 1)
    eye = (ii == jj).astype(jnp.float32)
    s_col = lax.dot_general(eye, s_row,
                            dimension_numbers=(((1,), (1,)), ((), ())),
                            preferred_element_type=jnp.float32)
    lane = lax.broadcasted_iota(jnp.int32, (W, W), 1)
    sub = lax.broadcasted_iota(jnp.int32, (W, W), 0)
    lt = (s_row < s_col).astype(jnp.float32)
    tie = ((s_row == s_col) & (lane < sub)).astype(jnp.float32)
    rank = jnp.sum(lt + tie, axis=1, keepdims=True)
    j = rank - float(W - K)
    jlane = lax.broadcasted_iota(jnp.int32, (W, K), 1).astype(jnp.float32)
    onehot = (j == jlane).astype(jnp.float32)
    csub = lax.broadcasted_iota(jnp.int32, (W, K), 0).astype(jnp.float32)
    ind_f = jnp.sum(onehot * csub, axis=0, keepdims=True)
    ind_ref[...] = ind_f.astype(jnp.int32)


_topk = pl.pallas_call(
    _topk_body,
    out_shape=jax.ShapeDtypeStruct((1, K), jnp.int32),
)


def _sc_gather_body(img_hbm, ind_hbm, out_hbm, ind_v,
                    in_v0, in_v1, out_v0, out_v1,
                    in_s0, in_s1, out_s0, out_s1):
    wid = lax.axis_index("s") * NC + lax.axis_index("c")
    base_row = wid * ROWS_PER_W
    in_bufs = (in_v0, in_v1)
    out_bufs = (out_v0, out_v1)
    in_sems = (in_s0, in_s1)
    out_sems = (out_s0, out_s1)

    pltpu.sync_copy(ind_hbm, ind_v)

    def start_in(c, b):
        row0 = base_row + c * R
        pltpu.async_copy(img_hbm.at[pl.ds(row0 * W, R * W)], in_bufs[b],
                         in_sems[b])

    def wait_in(b):
        pltpu.make_async_copy(img_hbm.at[pl.ds(0, R * W)], in_bufs[b],
                              in_sems[b]).wait()

    def start_out(c, b):
        row0 = base_row + c * R
        pltpu.async_copy(out_bufs[b], out_hbm.at[pl.ds(row0 * K, R * K)],
                         out_sems[b])

    def wait_out(b):
        pltpu.make_async_copy(out_bufs[b], out_hbm.at[pl.ds(0, R * K)],
                              out_sems[b]).wait()

    def gather_chunk(b):
        ivs0 = tuple(ind_v[pl.ds(16 * j, 16)] for j in range(K // 16))

        def row(r, ivs):
            for j in range(K // 16):
                out_bufs[b][pl.ds(r * K + 16 * j, 16)] = (
                    plsc.load_gather(in_bufs[b], [ivs[j]]))
            return tuple(iv + W for iv in ivs)

        plsc.parallel_loop(0, R, unroll=2, carry=ivs0)(row)

    start_in(0, 0)
    start_in(1, 1)

    def step(i, carry):
        for b in range(2):
            c = 2 * i + b
            wait_in(b)

            @pl.when(i > 0)
            def _():
                wait_out(b)

            gather_chunk(b)
            start_out(c, b)

            @pl.when(i < CHUNKS // 2 - 1)
            def _():
                start_in(c + 2, b)
        return carry

    lax.fori_loop(0, CHUNKS // 2, step, 0)
    wait_out(0)
    wait_out(1)


_sc_gather = functools.partial(
    pl.kernel,
    mesh=plsc.VectorSubcoreMesh(core_axis_name="c", subcore_axis_name="s"),
    out_type=jax.ShapeDtypeStruct((M * K,), jnp.float32),
    compiler_params=pltpu.CompilerParams(needs_layout_passes=False),
    scratch_types=[
        pltpu.VMEM((K,), jnp.int32),
        pltpu.VMEM((R * W,), jnp.float32),
        pltpu.VMEM((R * W,), jnp.float32),
        pltpu.VMEM((R * K,), jnp.float32),
        pltpu.VMEM((R * K,), jnp.float32),
        pltpu.SemaphoreType.DMA,
        pltpu.SemaphoreType.DMA,
        pltpu.SemaphoreType.DMA,
        pltpu.SemaphoreType.DMA,
    ],
)(_sc_gather_body)


def kernel(img, weight):
    w2 = weight[:, :, 0, 0]
    ind = _topk(w2).reshape(K)
    img_flat = img.reshape(M * W)
    out_flat = _sc_gather(img_flat, ind)
    return out_flat.reshape(8, 192, 224, K)


# speedup vs baseline: 8.9005x; 3.0567x over previous
import functools

import jax
import jax.numpy as jnp
from jax import lax
from jax.experimental import pallas as pl
from jax.experimental.pallas import tpu as pltpu
from jax.experimental.pallas import tpu_sc as plsc

W = 224
K = 128
M = 8 * 192 * 224
NC = 2
NS = 16
NW = NC * NS
ROWS_PER_W = M // NW
R = 112
CHUNKS = ROWS_PER_W // R


def _topk_body(w_ref, ind_ref):
    w = w_ref[...]
    s_row = jnp.sum(w, axis=0, keepdims=True)
    ii = lax.broadcasted_iota(jnp.int32, (W, W), 0)
    jj = lax.broadcasted_iota(jnp.int32, (W, W), 1)
    eye = (ii == jj).astype(jnp.float32)
    s_col = lax.dot_general(eye, s_row,
                            dimension_numbers=(((1,), (1,)), ((), ())),
                            preferred_element_type=jnp.float32)
    lane = lax.broadcasted_iota(jnp.int32, (W, W), 1)
    sub = lax.broadcasted_iota(jnp.int32, (W, W), 0)
    lt = (s_row < s_col).astype(jnp.float32)
    tie = ((s_row == s_col) & (lane < sub)).astype(jnp.float32)
    rank = jnp.sum(lt + tie, axis=1, keepdims=True)
    j = rank - float(W - K)
    jlane = lax.broadcasted_iota(jnp.int32, (W, K), 1).astype(jnp.float32)
    onehot = (j == jlane).astype(jnp.float32)
    csub = lax.broadcasted_iota(jnp.int32, (W, K), 0).astype(jnp.float32)
    ind_f = jnp.sum(onehot * csub, axis=0, keepdims=True)
    ind_ref[...] = ind_f.astype(jnp.int32)


_topk = pl.pallas_call(
    _topk_body,
    out_shape=jax.ShapeDtypeStruct((1, K), jnp.int32),
)


def _sc_gather_body(img_hbm, ind_hbm, out_hbm, ind_v,
                    in_v0, in_v1, out_v0, out_v1, sp_out,
                    in_s0, in_s1, x_s0, x_s1, out_s0, out_s1):
    sid = lax.axis_index("s")
    wid = sid * NC + lax.axis_index("c")
    base_row = wid * ROWS_PER_W
    in_bufs = (in_v0, in_v1)
    out_bufs = (out_v0, out_v1)
    in_sems = (in_s0, in_s1)
    x_sems = (x_s0, x_s1)
    out_sems = (out_s0, out_s1)

    pltpu.sync_copy(ind_hbm, ind_v)

    def start_in(c, b):
        row0 = base_row + c * R
        pltpu.async_copy(img_hbm.at[pl.ds(row0, R), pl.ds(0, 128)],
                         in_bufs[b].at[:, pl.ds(0, 128)], in_sems[b])
        pltpu.async_copy(img_hbm.at[pl.ds(row0, R), pl.ds(128, 96)],
                         in_bufs[b].at[:, pl.ds(128, 96)], in_sems[b])

    def wait_in(b):
        pltpu.make_async_copy(img_hbm.at[pl.ds(0, R), pl.ds(0, 128)],
                              in_bufs[b].at[:, pl.ds(0, 128)],
                              in_sems[b]).wait()
        pltpu.make_async_copy(img_hbm.at[pl.ds(0, R), pl.ds(128, 96)],
                              in_bufs[b].at[:, pl.ds(128, 96)],
                              in_sems[b]).wait()

    def start_x(b):
        pltpu.async_copy(out_bufs[b], sp_out.at[sid, b], x_sems[b])

    def wait_x(b):
        pltpu.make_async_copy(out_bufs[b], sp_out.at[sid, b],
                              x_sems[b]).wait()

    def start_out(c, b):
        row0 = base_row + c * R
        pltpu.async_copy(sp_out.at[sid, b], out_hbm.at[pl.ds(row0, R), :],
                         out_sems[b])

    def wait_out(b):
        pltpu.make_async_copy(sp_out.at[sid, b],
                              out_hbm.at[pl.ds(0, R), :],
                              out_sems[b]).wait()

    def gather_chunk(b):
        cvs = tuple(ind_v[pl.ds(16 * j, 16)] for j in range(K // 16))

        def row(r):
            rv = jnp.full((16,), 0, jnp.int32) + r
            for j in range(K // 16):
                out_bufs[b][r, pl.ds(16 * j, 16)] = (
                    plsc.load_gather(in_bufs[b], [rv, cvs[j]]))

        plsc.parallel_loop(0, R, unroll=4)(row)

    start_in(0, 0)
    start_in(1, 1)

    def step(i, carry):
        for b in range(2):
            c = 2 * i + b
            wait_in(b)

            @pl.when(i > 0)
            def _():
                wait_out(b)

            gather_chunk(b)
            start_x(b)

            @pl.when(i < CHUNKS // 2 - 1)
            def _():
                start_in(c + 2, b)

            wait_x(b)
            start_out(c, b)
        return carry

    lax.fori_loop(0, CHUNKS // 2, step, 0)
    wait_out(0)
    wait_out(1)


_sc_gather = functools.partial(
    pl.kernel,
    mesh=plsc.VectorSubcoreMesh(core_axis_name="c", subcore_axis_name="s"),
    out_type=jax.ShapeDtypeStruct((M, K), jnp.float32),
    compiler_params=pltpu.CompilerParams(needs_layout_passes=False),
    scratch_types=[
        pltpu.VMEM((K,), jnp.int32),
        pltpu.VMEM((R, W), jnp.float32),
        pltpu.VMEM((R, W), jnp.float32),
        pltpu.VMEM((R, K), jnp.float32),
        pltpu.VMEM((R, K), jnp.float32),
        pltpu.VMEM_SHARED((NS, 2, R, K), jnp.float32),
        pltpu.SemaphoreType.DMA,
        pltpu.SemaphoreType.DMA,
        pltpu.SemaphoreType.DMA,
        pltpu.SemaphoreType.DMA,
        pltpu.SemaphoreType.DMA,
        pltpu.SemaphoreType.DMA,
    ],
)(_sc_gather_body)


def kernel(img, weight):
    w2 = weight[:, :, 0, 0]
    ind = _topk(w2).reshape(K)
    img2 = img.reshape(M, W)
    out2 = _sc_gather(img2, ind)
    return out2.reshape(8, 192, 224, K)


# speedup vs baseline: 8.9915x; 1.0102x over previous
import functools

import jax
import jax.numpy as jnp
from jax import lax
from jax.experimental import pallas as pl
from jax.experimental.pallas import tpu as pltpu
from jax.experimental.pallas import tpu_sc as plsc

W = 224
K = 128
M = 8 * 192 * 224
NC = 2
NS = 16
NW = NC * NS
ROWS_PER_W = M // NW
R = 112
CHUNKS = ROWS_PER_W // R


def _topk_body(w_ref, ind_ref):
    w = w_ref[...]
    s_row = jnp.sum(w, axis=0, keepdims=True)
    ii = lax.broadcasted_iota(jnp.int32, (W, W), 0)
    jj = lax.broadcasted_iota(jnp.int32, (W, W), 1)
    eye = (ii == jj).astype(jnp.float32)
    s_col = lax.dot_general(eye, s_row,
                            dimension_numbers=(((1,), (1,)), ((), ())),
                            preferred_element_type=jnp.float32)
    lane = lax.broadcasted_iota(jnp.int32, (W, W), 1)
    sub = lax.broadcasted_iota(jnp.int32, (W, W), 0)
    lt = (s_row < s_col).astype(jnp.float32)
    tie = ((s_row == s_col) & (lane < sub)).astype(jnp.float32)
    rank = jnp.sum(lt + tie, axis=1, keepdims=True)
    j = rank - float(W - K)
    jlane = lax.broadcasted_iota(jnp.int32, (W, K), 1).astype(jnp.float32)
    onehot = (j == jlane).astype(jnp.float32)
    csub = lax.broadcasted_iota(jnp.int32, (W, K), 0).astype(jnp.float32)
    ind_f = jnp.sum(onehot * csub, axis=0, keepdims=True)
    ind_ref[...] = ind_f.astype(jnp.int32)


_topk = pl.pallas_call(
    _topk_body,
    out_shape=jax.ShapeDtypeStruct((1, K), jnp.int32),
)


def _sc_gather_body(img_hbm, ind_hbm, out_hbm, ind_v,
                    in_v0, in_v1, out_v0, out_v1, sp_out,
                    in_s0, in_s1, x_s0, x_s1, out_s0, out_s1):
    sid = lax.axis_index("s")
    wid = sid * NC + lax.axis_index("c")
    base_row = wid * ROWS_PER_W
    in_bufs = (in_v0, in_v1)
    out_bufs = (out_v0, out_v1)
    in_sems = (in_s0, in_s1)
    x_sems = (x_s0, x_s1)
    out_sems = (out_s0, out_s1)

    pltpu.sync_copy(ind_hbm, ind_v)

    def start_in(c, b):
        row0 = base_row + c * R
        pltpu.async_copy(img_hbm.at[pl.ds(row0, R), pl.ds(0, 128)],
                         in_bufs[b].at[:, pl.ds(0, 128)], in_sems[b])
        pltpu.async_copy(img_hbm.at[pl.ds(row0, R), pl.ds(128, 96)],
                         in_bufs[b].at[:, pl.ds(128, 96)], in_sems[b])

    def wait_in(b):
        pltpu.make_async_copy(img_hbm.at[pl.ds(0, R), pl.ds(0, 128)],
                              in_bufs[b].at[:, pl.ds(0, 128)],
                              in_sems[b]).wait()
        pltpu.make_async_copy(img_hbm.at[pl.ds(0, R), pl.ds(128, 96)],
                              in_bufs[b].at[:, pl.ds(128, 96)],
                              in_sems[b]).wait()

    def start_x(b):
        pltpu.async_copy(out_bufs[b], sp_out.at[sid, b], x_sems[b])

    def wait_x(b):
        pltpu.make_async_copy(out_bufs[b], sp_out.at[sid, b],
                              x_sems[b]).wait()

    def start_out(c, b):
        row0 = base_row + c * R
        pltpu.async_copy(sp_out.at[sid, b], out_hbm.at[pl.ds(row0, R), :],
                         out_sems[b])

    def wait_out(b):
        pltpu.make_async_copy(sp_out.at[sid, b],
                              out_hbm.at[pl.ds(0, R), :],
                              out_sems[b]).wait()

    def gather_chunk(b):
        cvs = tuple(ind_v[pl.ds(16 * j, 16)] for j in range(K // 16))

        def row(r):
            rv = jnp.full((16,), 0, jnp.int32) + r
            for j in range(K // 16):
                out_bufs[b][r, pl.ds(16 * j, 16)] = (
                    plsc.load_gather(in_bufs[b], [rv, cvs[j]]))

        plsc.parallel_loop(0, R, unroll=2)(row)

    start_in(0, 0)
    start_in(1, 1)

    def step(i, carry):
        for b in range(2):
            c = 2 * i + b
            wait_in(b)

            @pl.when(i > 0)
            def _():
                wait_out(b)

            gather_chunk(b)
            start_x(b)

            @pl.when(i < CHUNKS // 2 - 1)
            def _():
                start_in(c + 2, b)

            wait_x(b)
            start_out(c, b)
        return carry

    lax.fori_loop(0, CHUNKS // 2, step, 0)
    wait_out(0)
    wait_out(1)


_sc_gather = functools.partial(
    pl.kernel,
    mesh=plsc.VectorSubcoreMesh(core_axis_name="c", subcore_axis_name="s"),
    out_type=jax.ShapeDtypeStruct((M, K), jnp.float32),
    compiler_params=pltpu.CompilerParams(needs_layout_passes=False),
    scratch_types=[
        pltpu.VMEM((K,), jnp.int32),
        pltpu.VMEM((R, W), jnp.float32),
        pltpu.VMEM((R, W), jnp.float32),
        pltpu.VMEM((R, K), jnp.float32),
        pltpu.VMEM((R, K), jnp.float32),
        pltpu.VMEM_SHARED((NS, 2, R, K), jnp.float32),
        pltpu.SemaphoreType.DMA,
        pltpu.SemaphoreType.DMA,
        pltpu.SemaphoreType.DMA,
        pltpu.SemaphoreType.DMA,
        pltpu.SemaphoreType.DMA,
        pltpu.SemaphoreType.DMA,
    ],
)(_sc_gather_body)


def kernel(img, weight):
    w2 = weight[:, :, 0, 0]
    ind = _topk(w2).reshape(K)
    img2 = img.reshape(M, W)
    out2 = _sc_gather(img2, ind)
    return out2.reshape(8, 192, 224, K)
